# baseline (speedup 1.0000x reference)
import functools

import jax
import jax.numpy as jnp
from jax.experimental import pallas as pl
from jax.experimental.pallas import tpu as pltpu

E = 8
K = 2
T = 512
HT = 1024
DT = 512


def _gate_kernel(x_ref, wg_ref, bg_ref, wi_ref, ww_ref):
    x = x_ref[...]
    logits = jax.lax.dot_general(
        x, wg_ref[...], (((1,), (1,)), ((), ())),
        preferred_element_type=jnp.float32) + bg_ref[...]
    m1 = jnp.max(logits, axis=1, keepdims=True)
    i1 = jnp.argmax(logits, axis=1).astype(jnp.int32)
    iota = jax.lax.broadcasted_iota(jnp.int32, logits.shape, 1)
    masked = jnp.where(iota == i1[:, None], -jnp.inf, logits)
    m2 = jnp.max(masked, axis=1, keepdims=True)
    i2 = jnp.argmax(masked, axis=1).astype(jnp.int32)
    e2 = jnp.exp(m2 - m1)
    w1 = 1.0 / (1.0 + e2)
    w2 = e2 * w1
    wi_ref[...] = jnp.concatenate([i1[:, None], i2[:, None]], axis=1)
    ww_ref[...] = jnp.concatenate([w1, w2], axis=1)


def _ffn_kernel(blk_e_ref, nblk_ref, xs_ref, w1_ref, b1_ref, w2_ref, b2_ref,
                rw_ref, ys_ref, hact_ref, *, nht):
    g = pl.program_id(0)
    j = pl.program_id(1)

    @pl.when(g < nblk_ref[0])
    def _():
        @pl.when(j < nht)
        def _():
            hpre = jax.lax.dot_general(
                xs_ref[...], w1_ref[0].astype(jnp.bfloat16),
                (((1,), (1,)), ((), ())),
                preferred_element_type=jnp.float32)
            hact_ref[:, pl.ds(j * HT, HT)] = jnp.maximum(
                hpre + b1_ref[0], 0.0).astype(jnp.bfloat16)

        @pl.when(j >= nht)
        def _():
            d = j - nht
            y = jax.lax.dot_general(
                hact_ref[...], w2_ref[0].astype(jnp.bfloat16),
                (((1,), (1,)), ((), ())),
                preferred_element_type=jnp.float32)
            ys_ref[:, pl.ds(d * DT, DT)] = (
                (y + b2_ref[0]) * rw_ref[...]).astype(jnp.bfloat16)


def kernel(x, Wg, bg, W1, b1, W2, b2):
    Bn, S, D = x.shape
    H = W1.shape[1]
    xf = x.reshape(-1, D)
    n_tok = xf.shape[0]
    n_pair = n_tok * K
    G = n_pair // T + E
    nht = H // HT
    ndt = D // DT

    top_i, top_w = pl.pallas_call(
        _gate_kernel,
        out_shape=(jax.ShapeDtypeStruct((n_tok, K), jnp.int32),
                   jax.ShapeDtypeStruct((n_tok, K), jnp.float32)),
    )(xf, Wg, bg)

    pair_e = top_i.reshape(-1)
    pair_w = top_w.reshape(-1)
    onehot = (pair_e[:, None] == jnp.arange(E, dtype=jnp.int32)[None, :])
    onehot = onehot.astype(jnp.int32)
    incl = jnp.cumsum(onehot, axis=0)
    counts = incl[-1]
    rank = jnp.take_along_axis(incl - onehot, pair_e[:, None], axis=1)[:, 0]
    nblk = (counts + T - 1) // T
    cum_nblk = jnp.cumsum(nblk)
    blk_start = cum_nblk - nblk
    row_j = blk_start[pair_e] * T + rank
    gt = G * T
    tok_id = jnp.arange(n_pair, dtype=jnp.int32) // K
    row_token = jnp.zeros((gt,), jnp.int32).at[row_j].set(tok_id)
    row_weight = jnp.zeros((gt, 1), jnp.float32).at[row_j, 0].set(pair_w)
    blk_expert = jnp.minimum(
        jnp.searchsorted(cum_nblk, jnp.arange(G, dtype=jnp.int32),
                         side="right"),
        E - 1).astype(jnp.int32)
    total_blk = cum_nblk[E - 1].astype(jnp.int32)

    xs = jnp.take(xf.astype(jnp.bfloat16), row_token, axis=0)

    b1r = b1.reshape(E, 1, H)
    b2r = b2.reshape(E, 1, D)
    grid_spec = pltpu.PrefetchScalarGridSpec(
        num_scalar_prefetch=2,
        grid=(G, nht + ndt),
        in_specs=[
            pl.BlockSpec((T, D), lambda g, j, be, nb: (g, 0)),
            pl.BlockSpec(
                (1, HT, D),
                lambda g, j, be, nb: (be[g], jnp.minimum(j, nht - 1), 0)),
            pl.BlockSpec(
                (1, 1, HT),
                lambda g, j, be, nb: (be[g], 0, jnp.minimum(j, nht - 1))),
            pl.BlockSpec(
                (1, DT, H),
                lambda g, j, be, nb: (be[g], jnp.maximum(j - nht, 0), 0)),
            pl.BlockSpec(
                (1, 1, DT),
                lambda g, j, be, nb: (be[g], 0, jnp.maximum(j - nht, 0))),
            pl.BlockSpec((T, 1), lambda g, j, be, nb: (g, 0)),
        ],
        out_specs=pl.BlockSpec((T, D), lambda g, j, be, nb: (g, 0)),
        scratch_shapes=[pltpu.VMEM((T, H), jnp.bfloat16)],
    )
    ys = pl.pallas_call(
        functools.partial(_ffn_kernel, nht=nht),
        grid_spec=grid_spec,
        out_shape=jax.ShapeDtypeStruct((gt, D), jnp.bfloat16),
        compiler_params=pltpu.CompilerParams(
            dimension_semantics=("arbitrary", "arbitrary")),
    )(blk_expert, jnp.full((1,), total_blk, jnp.int32),
      xs, W1, b1r, W2, b2r, row_weight)

    pos = row_j.reshape(n_tok, K)
    y0 = jnp.take(ys, pos[:, 0], axis=0).astype(jnp.float32)
    y1 = jnp.take(ys, pos[:, 1], axis=0).astype(jnp.float32)
    return (y0 + y1).reshape(Bn, S, D)
